# baseline (speedup 1.0000x reference)
import math

import jax
import jax.numpy as jnp
from jax.experimental import pallas as pl
from jax.experimental.pallas import tpu as pltpu

_N_VOCAB = 100000
_PAD = 0
_SMOOTHING = 0.1
_CONF = 1.0 - _SMOOTHING
_S = _SMOOTHING / (_N_VOCAB - 2)
_C_ROW = (_N_VOCAB - 2) * _S * math.log(_S) + _CONF * math.log(_CONF)

_ROWS_BLK = 64


def _loss_kernel(t_smem, t_vmem, p_ref, out_ref):
    i = pl.program_id(0)
    p = p_ref[...]
    rowsum = jnp.sum(p, axis=1, keepdims=True)
    p0 = p[:, 0:1]

    g = jnp.zeros((1, 1), jnp.float32)
    for r in range(_ROWS_BLK):
        t_r = t_smem[r, 0]
        start = pl.multiple_of((t_r // 128) * 128, 128)
        win = p_ref[pl.ds(r, 1), pl.ds(start, 128)]
        lane = jax.lax.broadcasted_iota(jnp.int32, (1, 128), 1)
        val = jnp.sum(jnp.where(lane == (t_r % 128), win, 0.0),
                      axis=(0, 1), keepdims=True)
        g += jnp.where(t_r != _PAD, val, 0.0)

    valid = (t_vmem[...] != _PAD).astype(jnp.float32)
    contrib = valid * (_C_ROW - _S * (rowsum - p0))
    partial = jnp.sum(contrib, axis=(0, 1), keepdims=True)
    partial = partial - (_CONF - _S) * g

    @pl.when(i == 0)
    def _init():
        out_ref[...] = jnp.zeros_like(out_ref)

    out_ref[...] += partial


def kernel(predicts, target):
    n, v = predicts.shape
    t2 = target.reshape(n, 1).astype(jnp.int32)
    out = pl.pallas_call(
        _loss_kernel,
        grid=(n // _ROWS_BLK,),
        in_specs=[
            pl.BlockSpec((_ROWS_BLK, 1), lambda i: (i, 0),
                         memory_space=pltpu.SMEM),
            pl.BlockSpec((_ROWS_BLK, 1), lambda i: (i, 0)),
            pl.BlockSpec((_ROWS_BLK, v), lambda i: (i, 0)),
        ],
        out_specs=pl.BlockSpec((1, 1), lambda i: (0, 0)),
        out_shape=jax.ShapeDtypeStruct((1, 1), jnp.float32),
    )(t2, t2, predicts)
    return out[0, 0]
